# baseline (speedup 1.0000x reference)
import jax
import jax.numpy as jnp
from jax.experimental import pallas as pl
from jax.experimental.pallas import tpu as pltpu

N = 4096
F = 256
H = 128
TAU = 0.5
BM = 256
NI = N // BM
HC = N // 2


def _cl_gcn_kernel(x1_ref, x2_ref, adj1_ref, adj2_ref, clm_ref,
                   w11_ref, b11_ref, w12_ref, b12_ref,
                   w21_ref, b21_ref, w22_ref, b22_ref,
                   z1_ref, z2_ref, loss_ref,
                   adj_scr, sup1_scr, sup2_scr, s2a_scr, s2b_scr,
                   rs_scr, ws_scr, acc_ref):
    t = pl.program_id(0)

    @pl.when(t < NI)
    def _():
        i = t
        xb1 = x1_ref[...].astype(jnp.bfloat16)
        sup = jnp.dot(xb1, w11_ref[...], preferred_element_type=jnp.float32)
        sup1_scr[pl.ds(i * BM, BM), :] = sup.astype(jnp.bfloat16)
        xb2 = x2_ref[...].astype(jnp.bfloat16)
        sup = jnp.dot(xb2, w21_ref[...], preferred_element_type=jnp.float32)
        sup2_scr[pl.ds(i * BM, BM), :] = sup.astype(jnp.bfloat16)

    @pl.when(jnp.logical_and(t >= NI, t < 2 * NI))
    def _():
        i = t - NI
        ab = adj2_ref[...].astype(jnp.bfloat16)
        adj_scr[pl.ds(i * BM, BM), :] = ab
        acc = jnp.dot(ab, sup2_scr[...], preferred_element_type=jnp.float32)
        h = jnp.maximum(acc + b21_ref[...], 0.0)
        s2 = jnp.dot(h.astype(jnp.bfloat16), w22_ref[...],
                     preferred_element_type=jnp.float32)
        s2b_scr[pl.ds(i * BM, BM), :] = s2.astype(jnp.bfloat16)

    @pl.when(jnp.logical_and(t >= 2 * NI, t < 3 * NI))
    def _():
        i = t - 2 * NI
        a2 = adj_scr[pl.ds(i * BM, BM), :]
        z2 = jnp.dot(a2, s2b_scr[...],
                     preferred_element_type=jnp.float32) + b22_ref[...]
        z2_ref[...] = z2
        r2 = jax.lax.rsqrt(jnp.sum(z2 * z2, axis=1, keepdims=True))
        sup2_scr[pl.ds(i * BM, BM), :H] = (z2 * r2).astype(jnp.bfloat16)
        ab = adj1_ref[...].astype(jnp.bfloat16)
        adj_scr[pl.ds(i * BM, BM), :] = ab
        acc = jnp.dot(ab, sup1_scr[...], preferred_element_type=jnp.float32)
        h = jnp.maximum(acc + b11_ref[...], 0.0)
        s2 = jnp.dot(h.astype(jnp.bfloat16), w12_ref[...],
                     preferred_element_type=jnp.float32)
        s2a_scr[pl.ds(i * BM, BM), :] = s2.astype(jnp.bfloat16)

    @pl.when(t >= 3 * NI)
    def _():
        k = t - 3 * NI
        i = k // 2
        half = k % 2

        @pl.when(half == 0)
        def _():
            a1 = adj_scr[pl.ds(i * BM, BM), :]
            z1 = jnp.dot(a1, s2a_scr[...],
                         preferred_element_type=jnp.float32) + b12_ref[...]
            z1_ref[...] = z1
            r1 = jax.lax.rsqrt(jnp.sum(z1 * z1, axis=1, keepdims=True))
            sup1_scr[pl.ds(i * BM, BM), :H] = (
                z1 * r1 * (1.4426950408889634 / TAU)).astype(jnp.bfloat16)

        z1s = sup1_scr[pl.ds(i * BM, BM), :H]
        z2s = sup2_scr[pl.ds(half * HC, HC), :H]
        s = jax.lax.dot_general(z1s, z2s, (((1,), (1,)), ((), ())),
                                preferred_element_type=jnp.float32)
        pexp = jnp.exp2(s)
        prs = jnp.sum(pexp, axis=1, keepdims=True)
        pws = jnp.sum(pexp * clm_ref[...], axis=1, keepdims=True)

        @pl.when(half == 0)
        def _():
            rs_scr[...] = prs
            ws_scr[...] = pws

        @pl.when(half == 1)
        def _():
            rs = rs_scr[...] + prs
            ws = ws_scr[...] + pws
            part = jnp.sum(jnp.log(rs + 1e-8) - jnp.log(ws))

            @pl.when(i == 0)
            def _():
                acc_ref[0] = 0.0

            acc_ref[0] += part

            @pl.when(i == NI - 1)
            def _():
                loss_ref[...] = jnp.full((1, 1), acc_ref[0] * (1.0 / N),
                                         dtype=jnp.float32)


def _cl_gcn(x1, adj1, x2, adj2, clm,
            W11, b11, W12, b12, W21, b21, W22, b22):
    ni = NI
    z1, z2, loss = pl.pallas_call(
        _cl_gcn_kernel,
        grid=(5 * ni,),
        in_specs=[
            pl.BlockSpec((BM, F),
                         lambda t: (jnp.where(t < ni, t, ni - 1), 0)),
            pl.BlockSpec((BM, F),
                         lambda t: (jnp.where(t < ni, t, ni - 1), 0)),
            pl.BlockSpec((BM, N),
                         lambda t: (jnp.where(t < 2 * ni, 0,
                                              jnp.where(t < 3 * ni, t - 2 * ni,
                                                        ni - 1)), 0)),
            pl.BlockSpec((BM, N),
                         lambda t: (jnp.where(t < ni, 0,
                                              jnp.where(t < 2 * ni, t - ni,
                                                        ni - 1)), 0)),
            pl.BlockSpec((BM, HC),
                         lambda t: (jnp.where(t < 3 * ni, 0,
                                              (t - 3 * ni) // 2),
                                    jnp.where(t < 3 * ni, 0,
                                              (t - 3 * ni) % 2))),
            pl.BlockSpec((F, F), lambda t: (0, 0)),
            pl.BlockSpec((1, F), lambda t: (0, 0)),
            pl.BlockSpec((F, H), lambda t: (0, 0)),
            pl.BlockSpec((1, H), lambda t: (0, 0)),
            pl.BlockSpec((F, F), lambda t: (0, 0)),
            pl.BlockSpec((1, F), lambda t: (0, 0)),
            pl.BlockSpec((F, H), lambda t: (0, 0)),
            pl.BlockSpec((1, H), lambda t: (0, 0)),
        ],
        out_specs=[
            pl.BlockSpec((BM, H),
                         lambda t: (jnp.where(t < 3 * ni, 0,
                                              (t - 3 * ni) // 2), 0)),
            pl.BlockSpec((BM, H),
                         lambda t: (jnp.where(t < 2 * ni, 0,
                                              jnp.where(t < 3 * ni, t - 2 * ni,
                                                        ni - 1)), 0)),
            pl.BlockSpec((1, 1), lambda t: (0, 0)),
        ],
        out_shape=[
            jax.ShapeDtypeStruct((N, H), jnp.float32),
            jax.ShapeDtypeStruct((N, H), jnp.float32),
            jax.ShapeDtypeStruct((1, 1), jnp.float32),
        ],
        scratch_shapes=[
            pltpu.VMEM((N, N), jnp.bfloat16),
            pltpu.VMEM((N, F), jnp.bfloat16),
            pltpu.VMEM((N, F), jnp.bfloat16),
            pltpu.VMEM((N, H), jnp.bfloat16),
            pltpu.VMEM((N, H), jnp.bfloat16),
            pltpu.VMEM((BM, 1), jnp.float32),
            pltpu.VMEM((BM, 1), jnp.float32),
            pltpu.SMEM((1,), jnp.float32),
        ],
        compiler_params=pltpu.CompilerParams(
            vmem_limit_bytes=63 * 1024 * 1024,
        ),
    )(x1, x2, adj1, adj2, clm,
      W11.astype(jnp.bfloat16), b11.reshape(1, F),
      W12.astype(jnp.bfloat16), b12.reshape(1, H),
      W21.astype(jnp.bfloat16), b21.reshape(1, F),
      W22.astype(jnp.bfloat16), b22.reshape(1, H))
    return z1, z2, loss.reshape(())


def kernel(x1, adj1, x2, adj2, clm, W11, b11, W12, b12, W21, b21, W22, b22):
    z1, z2, loss = _cl_gcn(x1, adj1, x2, adj2, clm,
                           W11, b11, W12, b12, W21, b21, W22, b22)
    return (z1, z2, loss)


# speedup vs baseline: 1.0469x; 1.0469x over previous
import jax
import jax.numpy as jnp
from jax.experimental import pallas as pl
from jax.experimental.pallas import tpu as pltpu

N = 4096
F = 256
H = 128
TAU = 0.5
HC = N // 2
LOG2E = 1.4426950408889634


def _towers_kernel(x1_ref, x2_ref, adj1l_ref, adj1r_ref, adj2l_ref, adj2r_ref,
                   w11_ref, b11_ref, w12_ref, b12_ref,
                   w21_ref, b21_ref, w22_ref, b22_ref,
                   z1_ref, z2_ref,
                   adj_scr, sup1_scr, sup2_scr, s2a_scr, s2b_scr):
    p = pl.program_id(0)
    i = pl.program_id(1)
    bm = z1_ref.shape[0]

    @pl.when(p == 0)
    def _():
        xb1 = x1_ref[...].astype(jnp.bfloat16)
        sup = jnp.dot(xb1, w11_ref[...], preferred_element_type=jnp.float32)
        sup1_scr[pl.ds(i * bm, bm), :] = sup.astype(jnp.bfloat16)
        xb2 = x2_ref[...].astype(jnp.bfloat16)
        sup = jnp.dot(xb2, w21_ref[...], preferred_element_type=jnp.float32)
        sup2_scr[pl.ds(i * bm, bm), :] = sup.astype(jnp.bfloat16)

    @pl.when(p == 1)
    def _():
        abl = adj1l_ref[...].astype(jnp.bfloat16)
        abr = adj1r_ref[...].astype(jnp.bfloat16)
        adj_scr[pl.ds(i * bm, bm), :HC] = abl
        adj_scr[pl.ds(i * bm, bm), HC:] = abr
        acc = (jnp.dot(abl, sup1_scr[:HC, :],
                       preferred_element_type=jnp.float32) +
               jnp.dot(abr, sup1_scr[HC:, :],
                       preferred_element_type=jnp.float32))
        h = jnp.maximum(acc + b11_ref[...], 0.0)
        s2 = jnp.dot(h.astype(jnp.bfloat16), w12_ref[...],
                     preferred_element_type=jnp.float32)
        s2a_scr[pl.ds(i * bm, bm), :] = s2.astype(jnp.bfloat16)

    @pl.when(p == 2)
    def _():
        a1 = adj_scr[pl.ds(i * bm, bm), :]
        z1_ref[...] = jnp.dot(a1, s2a_scr[...],
                              preferred_element_type=jnp.float32) + b12_ref[...]
        abl = adj2l_ref[...].astype(jnp.bfloat16)
        abr = adj2r_ref[...].astype(jnp.bfloat16)
        adj_scr[pl.ds(i * bm, bm), :HC] = abl
        adj_scr[pl.ds(i * bm, bm), HC:] = abr
        acc = (jnp.dot(abl, sup2_scr[:HC, :],
                       preferred_element_type=jnp.float32) +
               jnp.dot(abr, sup2_scr[HC:, :],
                       preferred_element_type=jnp.float32))
        h = jnp.maximum(acc + b21_ref[...], 0.0)
        s2 = jnp.dot(h.astype(jnp.bfloat16), w22_ref[...],
                     preferred_element_type=jnp.float32)
        s2b_scr[pl.ds(i * bm, bm), :] = s2.astype(jnp.bfloat16)

    @pl.when(p == 3)
    def _():
        a2 = adj_scr[pl.ds(i * bm, bm), :]
        z2_ref[...] = jnp.dot(a2, s2b_scr[...],
                              preferred_element_type=jnp.float32) + b22_ref[...]


def _towers(x1, x2, adj1, adj2,
            W11, b11, W12, b12, W21, b21, W22, b22, bm=256):
    ni = N // bm

    def _stream_map(phase):
        def _m(p, i):
            return (jnp.where(p < phase, 0,
                              jnp.where(p == phase, i, ni - 1)),)
        return _m

    z1, z2 = pl.pallas_call(
        _towers_kernel,
        grid=(4, ni),
        in_specs=[
            pl.BlockSpec((bm, F),
                         lambda p, i: (jnp.where(p == 0, i, ni - 1), 0)),
            pl.BlockSpec((bm, F),
                         lambda p, i: (jnp.where(p == 0, i, ni - 1), 0)),
            pl.BlockSpec((bm, HC),
                         lambda p, i: (_stream_map(1)(p, i)[0], 0)),
            pl.BlockSpec((bm, HC),
                         lambda p, i: (_stream_map(1)(p, i)[0], 1)),
            pl.BlockSpec((bm, HC),
                         lambda p, i: (_stream_map(2)(p, i)[0], 0)),
            pl.BlockSpec((bm, HC),
                         lambda p, i: (_stream_map(2)(p, i)[0], 1)),
            pl.BlockSpec((F, F), lambda p, i: (0, 0)),
            pl.BlockSpec((1, F), lambda p, i: (0, 0)),
            pl.BlockSpec((F, H), lambda p, i: (0, 0)),
            pl.BlockSpec((1, H), lambda p, i: (0, 0)),
            pl.BlockSpec((F, F), lambda p, i: (0, 0)),
            pl.BlockSpec((1, F), lambda p, i: (0, 0)),
            pl.BlockSpec((F, H), lambda p, i: (0, 0)),
            pl.BlockSpec((1, H), lambda p, i: (0, 0)),
        ],
        out_specs=[
            pl.BlockSpec((bm, H),
                         lambda p, i: (jnp.where(p < 2, 0,
                                                 jnp.where(p == 2, i,
                                                           ni - 1)), 0)),
            pl.BlockSpec((bm, H),
                         lambda p, i: (jnp.where(p < 3, 0, i), 0)),
        ],
        out_shape=[
            jax.ShapeDtypeStruct((N, H), jnp.float32),
            jax.ShapeDtypeStruct((N, H), jnp.float32),
        ],
        scratch_shapes=[
            pltpu.VMEM((N, N), jnp.bfloat16),
            pltpu.VMEM((N, F), jnp.bfloat16),
            pltpu.VMEM((N, F), jnp.bfloat16),
            pltpu.VMEM((N, H), jnp.bfloat16),
            pltpu.VMEM((N, H), jnp.bfloat16),
        ],
        compiler_params=pltpu.CompilerParams(
            vmem_limit_bytes=63 * 1024 * 1024,
        ),
    )(x1, x2, adj1, adj1, adj2, adj2,
      W11.astype(jnp.bfloat16), b11.reshape(1, F),
      W12.astype(jnp.bfloat16), b12.reshape(1, H),
      W21.astype(jnp.bfloat16), b21.reshape(1, F),
      W22.astype(jnp.bfloat16), b22.reshape(1, H))
    return z1, z2


def _sim_kernel(z1_ref, z2_ref, clml_ref, clmr_ref, loss_ref, acc_ref):
    i = pl.program_id(0)
    z1 = z1_ref[...]
    z2 = z2_ref[...]
    r1 = jax.lax.rsqrt(jnp.sum(z1 * z1, axis=1, keepdims=True)) * (LOG2E / TAU)
    r2 = jax.lax.rsqrt(jnp.sum(z2 * z2, axis=1, keepdims=True))
    s = jax.lax.dot_general(z1, z2, (((1,), (1,)), ((), ())),
                            preferred_element_type=jnp.float32)
    p = jnp.exp2(s * r1 * r2.reshape(1, -1))
    rs = jnp.sum(p, axis=1, keepdims=True)
    ws = (jnp.sum(p[:, :HC] * clml_ref[...], axis=1, keepdims=True) +
          jnp.sum(p[:, HC:] * clmr_ref[...], axis=1, keepdims=True))
    part = jnp.sum(jnp.log(rs + 1e-8) - jnp.log(ws))

    @pl.when(i == 0)
    def _():
        acc_ref[0] = 0.0

    acc_ref[0] += part

    @pl.when(i == pl.num_programs(0) - 1)
    def _():
        loss_ref[...] = jnp.full((1, 1), acc_ref[0] * (1.0 / N),
                                 dtype=jnp.float32)


def _sim_loss(z1, z2, clm, bm=512):
    loss = pl.pallas_call(
        _sim_kernel,
        grid=(N // bm,),
        in_specs=[
            pl.BlockSpec((bm, H), lambda i: (i, 0)),
            pl.BlockSpec((N, H), lambda i: (0, 0)),
            pl.BlockSpec((bm, HC), lambda i: (i, 0)),
            pl.BlockSpec((bm, HC), lambda i: (i, 1)),
        ],
        out_specs=pl.BlockSpec((1, 1), lambda i: (0, 0)),
        out_shape=jax.ShapeDtypeStruct((1, 1), jnp.float32),
        scratch_shapes=[pltpu.SMEM((1,), jnp.float32)],
    )(z1, z2, clm, clm)
    return loss.reshape(())


def kernel(x1, adj1, x2, adj2, clm, W11, b11, W12, b12, W21, b21, W22, b22):
    z1, z2 = _towers(x1, x2, adj1, adj2,
                     W11, b11, W12, b12, W21, b21, W22, b22)
    loss = _sim_loss(z1, z2, clm)
    return (z1, z2, loss)


# speedup vs baseline: 1.0609x; 1.0134x over previous
import jax
import jax.numpy as jnp
from jax.experimental import pallas as pl
from jax.experimental.pallas import tpu as pltpu

N = 4096
F = 256
H = 128
TAU = 0.5
HC = N // 2
BM = 256
NI = N // BM
LOG2E = 1.4426950408889634


def _towers_kernel(x1_ref, x2_ref, adj1_ref, adj2_ref,
                   w11_ref, b11_ref, w12_ref, b12_ref,
                   w21_ref, b21_ref, w22_ref, b22_ref,
                   z1_ref, z2_ref,
                   adj_scr, sup1_scr, sup2_scr, s2a_scr, s2b_scr,
                   buf0, buf1, x1_buf, x2_buf, sems, xsems):
    bufs = (buf0, buf1)

    def adj_copy(adj_ref, k, parity):
        return pltpu.make_async_copy(
            adj_ref.at[pl.ds(k * BM, BM), :], bufs[parity], sems.at[parity])

    adj_copy(adj1_ref, 0, 0).start()
    cx1 = pltpu.make_async_copy(x1_ref, x1_buf, xsems.at[0])
    cx2 = pltpu.make_async_copy(x2_ref, x2_buf, xsems.at[1])
    cx1.start()
    cx2.start()

    cx1.wait()
    sup = jnp.dot(x1_buf[...].astype(jnp.bfloat16), w11_ref[...],
                  preferred_element_type=jnp.float32)
    sup1_scr[...] = sup.astype(jnp.bfloat16)
    cx2.wait()
    sup = jnp.dot(x2_buf[...].astype(jnp.bfloat16), w21_ref[...],
                  preferred_element_type=jnp.float32)
    sup2_scr[...] = sup.astype(jnp.bfloat16)

    for k in range(NI):
        if k + 1 < NI:
            adj_copy(adj1_ref, k + 1, (k + 1) % 2).start()
        adj_copy(adj1_ref, k, k % 2).wait()
        ab = bufs[k % 2][...].astype(jnp.bfloat16)
        adj_scr[pl.ds(k * BM, BM), :] = ab
        acc = jnp.dot(ab, sup1_scr[...], preferred_element_type=jnp.float32)
        h = jnp.maximum(acc + b11_ref[...], 0.0)
        s2 = jnp.dot(h.astype(jnp.bfloat16), w12_ref[...],
                     preferred_element_type=jnp.float32)
        s2a_scr[pl.ds(k * BM, BM), :] = s2.astype(jnp.bfloat16)

    adj_copy(adj2_ref, 0, 0).start()
    for k in range(NI):
        if k + 1 < NI:
            adj_copy(adj2_ref, k + 1, (k + 1) % 2).start()
        a1 = adj_scr[pl.ds(k * BM, BM), :]
        z1_ref[pl.ds(k * BM, BM), :] = jnp.dot(
            a1, s2a_scr[...], preferred_element_type=jnp.float32) + b12_ref[...]
        adj_copy(adj2_ref, k, k % 2).wait()
        ab = bufs[k % 2][...].astype(jnp.bfloat16)
        adj_scr[pl.ds(k * BM, BM), :] = ab
        acc = jnp.dot(ab, sup2_scr[...], preferred_element_type=jnp.float32)
        h = jnp.maximum(acc + b21_ref[...], 0.0)
        s2 = jnp.dot(h.astype(jnp.bfloat16), w22_ref[...],
                     preferred_element_type=jnp.float32)
        s2b_scr[pl.ds(k * BM, BM), :] = s2.astype(jnp.bfloat16)

    for k in range(NI):
        a2 = adj_scr[pl.ds(k * BM, BM), :]
        z2_ref[pl.ds(k * BM, BM), :] = jnp.dot(
            a2, s2b_scr[...], preferred_element_type=jnp.float32) + b22_ref[...]


def _towers(x1, x2, adj1, adj2,
            W11, b11, W12, b12, W21, b21, W22, b22):
    z1, z2 = pl.pallas_call(
        _towers_kernel,
        in_specs=[
            pl.BlockSpec(memory_space=pl.ANY),
            pl.BlockSpec(memory_space=pl.ANY),
            pl.BlockSpec(memory_space=pl.ANY),
            pl.BlockSpec(memory_space=pl.ANY),
            pl.BlockSpec((F, F), lambda: (0, 0)),
            pl.BlockSpec((1, F), lambda: (0, 0)),
            pl.BlockSpec((F, H), lambda: (0, 0)),
            pl.BlockSpec((1, H), lambda: (0, 0)),
            pl.BlockSpec((F, F), lambda: (0, 0)),
            pl.BlockSpec((1, F), lambda: (0, 0)),
            pl.BlockSpec((F, H), lambda: (0, 0)),
            pl.BlockSpec((1, H), lambda: (0, 0)),
        ],
        out_specs=[
            pl.BlockSpec((N, H), lambda: (0, 0)),
            pl.BlockSpec((N, H), lambda: (0, 0)),
        ],
        out_shape=[
            jax.ShapeDtypeStruct((N, H), jnp.float32),
            jax.ShapeDtypeStruct((N, H), jnp.float32),
        ],
        scratch_shapes=[
            pltpu.VMEM((N, N), jnp.bfloat16),
            pltpu.VMEM((N, F), jnp.bfloat16),
            pltpu.VMEM((N, F), jnp.bfloat16),
            pltpu.VMEM((N, H), jnp.bfloat16),
            pltpu.VMEM((N, H), jnp.bfloat16),
            pltpu.VMEM((BM, N), jnp.float32),
            pltpu.VMEM((BM, N), jnp.float32),
            pltpu.VMEM((N, F), jnp.float32),
            pltpu.VMEM((N, F), jnp.float32),
            pltpu.SemaphoreType.DMA((2,)),
            pltpu.SemaphoreType.DMA((2,)),
        ],
        compiler_params=pltpu.CompilerParams(
            vmem_limit_bytes=63 * 1024 * 1024,
        ),
    )(x1, x2, adj1, adj2,
      W11.astype(jnp.bfloat16), b11.reshape(1, F),
      W12.astype(jnp.bfloat16), b12.reshape(1, H),
      W21.astype(jnp.bfloat16), b21.reshape(1, F),
      W22.astype(jnp.bfloat16), b22.reshape(1, H))
    return z1, z2


def _sim_kernel(z1_ref, z2_ref, clml_ref, clmr_ref, loss_ref, acc_ref):
    i = pl.program_id(0)
    z1 = z1_ref[...]
    z2 = z2_ref[...]
    r1 = jax.lax.rsqrt(jnp.sum(z1 * z1, axis=1, keepdims=True)) * (LOG2E / TAU)
    r2 = jax.lax.rsqrt(jnp.sum(z2 * z2, axis=1, keepdims=True))
    s = jax.lax.dot_general(z1, z2, (((1,), (1,)), ((), ())),
                            preferred_element_type=jnp.float32)
    p = jnp.exp2(s * r1 * r2.reshape(1, -1))
    rs = jnp.sum(p, axis=1, keepdims=True)
    ws = (jnp.sum(p[:, :HC] * clml_ref[...], axis=1, keepdims=True) +
          jnp.sum(p[:, HC:] * clmr_ref[...], axis=1, keepdims=True))
    part = jnp.sum(jnp.log(rs + 1e-8) - jnp.log(ws))

    @pl.when(i == 0)
    def _():
        acc_ref[0] = 0.0

    acc_ref[0] += part

    @pl.when(i == pl.num_programs(0) - 1)
    def _():
        loss_ref[...] = jnp.full((1, 1), acc_ref[0] * (1.0 / N),
                                 dtype=jnp.float32)


def _sim_loss(z1, z2, clm, bm=512):
    loss = pl.pallas_call(
        _sim_kernel,
        grid=(N // bm,),
        in_specs=[
            pl.BlockSpec((bm, H), lambda i: (i, 0)),
            pl.BlockSpec((N, H), lambda i: (0, 0)),
            pl.BlockSpec((bm, HC), lambda i: (i, 0)),
            pl.BlockSpec((bm, HC), lambda i: (i, 1)),
        ],
        out_specs=pl.BlockSpec((1, 1), lambda i: (0, 0)),
        out_shape=jax.ShapeDtypeStruct((1, 1), jnp.float32),
        scratch_shapes=[pltpu.SMEM((1,), jnp.float32)],
    )(z1, z2, clm, clm)
    return loss.reshape(())


def kernel(x1, adj1, x2, adj2, clm, W11, b11, W12, b12, W21, b21, W22, b22):
    z1, z2 = _towers(x1, x2, adj1, adj2,
                     W11, b11, W12, b12, W21, b21, W22, b22)
    loss = _sim_loss(z1, z2, clm)
    return (z1, z2, loss)


# speedup vs baseline: 1.1456x; 1.0798x over previous
import jax
import jax.numpy as jnp
from jax.experimental import pallas as pl
from jax.experimental.pallas import tpu as pltpu

N = 4096
F = 256
H = 128
TAU = 0.5
HC = N // 2
BM = 256
NI = N // BM
NB = 3
LOG2E = 1.4426950408889634


def _cl_gcn_kernel(x1_ref, x2_ref, adj1_ref, adj2_ref, clm_ref,
                   w11_ref, b11_ref, w12_ref, b12_ref,
                   w21_ref, b21_ref, w22_ref, b22_ref,
                   z1_ref, z2_ref, loss_ref,
                   adj_scr, sup1_scr, sup2_scr, s2a_scr, s2b_scr,
                   buf0, buf1, buf2, sems, acc_ref):
    bufs = (buf0, buf1, buf2)

    def x_task(x_hbm, k):
        def start(b, sem):
            pltpu.make_async_copy(
                x_hbm.at[pl.ds(k * BM, BM), :], b.at[:, :F], sem).start()

        def wait(b, sem):
            pltpu.make_async_copy(
                x_hbm.at[pl.ds(k * BM, BM), :], b.at[:, :F], sem).wait()
        return start, wait

    def row_task(a_hbm, k):
        def start(b, sem):
            pltpu.make_async_copy(
                a_hbm.at[pl.ds(k * BM, BM), :], b, sem).start()

        def wait(b, sem):
            pltpu.make_async_copy(
                a_hbm.at[pl.ds(k * BM, BM), :], b, sem).wait()
        return start, wait

    tasks = ([x_task(x1_ref, k) for k in range(NI)] +
             [x_task(x2_ref, k) for k in range(NI)] +
             [row_task(adj1_ref, k) for k in range(NI)] +
             [row_task(adj2_ref, k) for k in range(NI)] +
             [row_task(clm_ref, k) for k in range(NI)])
    T = len(tasks)

    def start_task(t):
        if t < T:
            tasks[t][0](bufs[t % NB], sems.at[t % NB])

    def wait_task(t):
        tasks[t][1](bufs[t % NB], sems.at[t % NB])

    for t in range(NB):
        start_task(t)

    t = 0
    for tower in range(2):
        sup_scr = sup1_scr if tower == 0 else sup2_scr
        w_ref = w11_ref if tower == 0 else w21_ref
        for k in range(NI):
            wait_task(t)
            xb = bufs[t % NB][:, :F].astype(jnp.bfloat16)
            sup = jnp.dot(xb, w_ref[...], preferred_element_type=jnp.float32)
            sup_scr[pl.ds(k * BM, BM), :] = sup.astype(jnp.bfloat16)
            start_task(t + NB)
            t += 1

    for k in range(NI):
        wait_task(t)
        ab = bufs[t % NB][...].astype(jnp.bfloat16)
        adj_scr[pl.ds(k * BM, BM), :] = ab
        acc = jnp.dot(ab, sup1_scr[...], preferred_element_type=jnp.float32)
        h = jnp.maximum(acc + b11_ref[...], 0.0)
        s2 = jnp.dot(h.astype(jnp.bfloat16), w12_ref[...],
                     preferred_element_type=jnp.float32)
        s2a_scr[pl.ds(k * BM, BM), :] = s2.astype(jnp.bfloat16)
        start_task(t + NB)
        t += 1

    for k in range(NI):
        a1 = adj_scr[pl.ds(k * BM, BM), :]
        z1 = jnp.dot(a1, s2a_scr[...],
                     preferred_element_type=jnp.float32) + b12_ref[...]
        z1_ref[pl.ds(k * BM, BM), :] = z1
        r1 = jax.lax.rsqrt(jnp.sum(z1 * z1, axis=1, keepdims=True))
        sup1_scr[pl.ds(k * BM, BM), :H] = (z1 * r1 * (LOG2E / TAU)
                                           ).astype(jnp.bfloat16)
        wait_task(t)
        ab = bufs[t % NB][...].astype(jnp.bfloat16)
        adj_scr[pl.ds(k * BM, BM), :] = ab
        acc = jnp.dot(ab, sup2_scr[...], preferred_element_type=jnp.float32)
        h = jnp.maximum(acc + b21_ref[...], 0.0)
        s2 = jnp.dot(h.astype(jnp.bfloat16), w22_ref[...],
                     preferred_element_type=jnp.float32)
        s2b_scr[pl.ds(k * BM, BM), :] = s2.astype(jnp.bfloat16)
        start_task(t + NB)
        t += 1

    for k in range(NI):
        a2 = adj_scr[pl.ds(k * BM, BM), :]
        z2 = jnp.dot(a2, s2b_scr[...],
                     preferred_element_type=jnp.float32) + b22_ref[...]
        z2_ref[pl.ds(k * BM, BM), :] = z2
        r2 = jax.lax.rsqrt(jnp.sum(z2 * z2, axis=1, keepdims=True))
        sup2_scr[pl.ds(k * BM, BM), :H] = (z2 * r2).astype(jnp.bfloat16)

    for k in range(NI):
        wait_task(t)
        clm = bufs[t % NB]
        z1s = sup1_scr[pl.ds(k * BM, BM), :H]
        rs = jnp.zeros((BM, 1), dtype=jnp.float32)
        ws = jnp.zeros((BM, 1), dtype=jnp.float32)
        for half in range(2):
            z2s = sup2_scr[pl.ds(half * HC, HC), :H]
            s = jax.lax.dot_general(z1s, z2s, (((1,), (1,)), ((), ())),
                                    preferred_element_type=jnp.float32)
            pexp = jnp.exp2(s)
            rs = rs + jnp.sum(pexp, axis=1, keepdims=True)
            ws = ws + jnp.sum(pexp * clm[:, half * HC:(half + 1) * HC],
                              axis=1, keepdims=True)
        part = jnp.sum(jnp.log(rs + 1e-8) - jnp.log(ws))
        if k == 0:
            acc_ref[0] = part
        else:
            acc_ref[0] += part
        start_task(t + NB)
        t += 1

    loss_ref[...] = jnp.full((1, 1), acc_ref[0] * (1.0 / N),
                             dtype=jnp.float32)


def _cl_gcn(x1, adj1, x2, adj2, clm,
            W11, b11, W12, b12, W21, b21, W22, b22):
    z1, z2, loss = pl.pallas_call(
        _cl_gcn_kernel,
        in_specs=[
            pl.BlockSpec(memory_space=pl.ANY),
            pl.BlockSpec(memory_space=pl.ANY),
            pl.BlockSpec(memory_space=pl.ANY),
            pl.BlockSpec(memory_space=pl.ANY),
            pl.BlockSpec(memory_space=pl.ANY),
            pl.BlockSpec((F, F), lambda: (0, 0)),
            pl.BlockSpec((1, F), lambda: (0, 0)),
            pl.BlockSpec((F, H), lambda: (0, 0)),
            pl.BlockSpec((1, H), lambda: (0, 0)),
            pl.BlockSpec((F, F), lambda: (0, 0)),
            pl.BlockSpec((1, F), lambda: (0, 0)),
            pl.BlockSpec((F, H), lambda: (0, 0)),
            pl.BlockSpec((1, H), lambda: (0, 0)),
        ],
        out_specs=[
            pl.BlockSpec((N, H), lambda: (0, 0)),
            pl.BlockSpec((N, H), lambda: (0, 0)),
            pl.BlockSpec((1, 1), lambda: (0, 0)),
        ],
        out_shape=[
            jax.ShapeDtypeStruct((N, H), jnp.float32),
            jax.ShapeDtypeStruct((N, H), jnp.float32),
            jax.ShapeDtypeStruct((1, 1), jnp.float32),
        ],
        scratch_shapes=[
            pltpu.VMEM((N, N), jnp.bfloat16),
            pltpu.VMEM((N, F), jnp.bfloat16),
            pltpu.VMEM((N, F), jnp.bfloat16),
            pltpu.VMEM((N, H), jnp.bfloat16),
            pltpu.VMEM((N, H), jnp.bfloat16),
            pltpu.VMEM((BM, N), jnp.float32),
            pltpu.VMEM((BM, N), jnp.float32),
            pltpu.VMEM((BM, N), jnp.float32),
            pltpu.SemaphoreType.DMA((NB,)),
            pltpu.SMEM((1,), jnp.float32),
        ],
        compiler_params=pltpu.CompilerParams(
            vmem_limit_bytes=63 * 1024 * 1024,
        ),
    )(x1, x2, adj1, adj2, clm,
      W11.astype(jnp.bfloat16), b11.reshape(1, F),
      W12.astype(jnp.bfloat16), b12.reshape(1, H),
      W21.astype(jnp.bfloat16), b21.reshape(1, F),
      W22.astype(jnp.bfloat16), b22.reshape(1, H))
    return z1, z2, loss.reshape(())


def kernel(x1, adj1, x2, adj2, clm, W11, b11, W12, b12, W21, b21, W22, b22):
    z1, z2, loss = _cl_gcn(x1, adj1, x2, adj2, clm,
                           W11, b11, W12, b12, W21, b21, W22, b22)
    return (z1, z2, loss)


# speedup vs baseline: 1.1818x; 1.0316x over previous
import jax
import jax.numpy as jnp
from jax.experimental import pallas as pl
from jax.experimental.pallas import tpu as pltpu

N = 4096
F = 256
H = 128
TAU = 0.5
HC = N // 2
BM = 256
NI = N // BM
NB = 4
LOG2E = 1.4426950408889634


def _cl_gcn_kernel(x1_ref, x2_ref, adj1_ref, adj2_ref, clm_ref,
                   w11_ref, b11_ref, w12_ref, b12_ref,
                   w21_ref, b21_ref, w22_ref, b22_ref,
                   z1_ref, z2_ref, loss_ref,
                   adj_scr, sup1_scr, sup2_scr, s2a_scr, s2b_scr,
                   buf0, buf1, buf2, buf3, sems, acc_ref):
    bufs = (buf0, buf1, buf2, buf3)

    def x_task(x_hbm, k):
        def start(b, sem):
            pltpu.make_async_copy(
                x_hbm.at[pl.ds(k * BM, BM), :], b.at[:, :F], sem).start()

        def wait(b, sem):
            pltpu.make_async_copy(
                x_hbm.at[pl.ds(k * BM, BM), :], b.at[:, :F], sem).wait()
        return start, wait

    def row_task(a_hbm, k):
        def start(b, sem):
            pltpu.make_async_copy(
                a_hbm.at[pl.ds(k * BM, BM), :], b, sem).start()

        def wait(b, sem):
            pltpu.make_async_copy(
                a_hbm.at[pl.ds(k * BM, BM), :], b, sem).wait()
        return start, wait

    tasks = ([x_task(x1_ref, k) for k in range(NI)] +
             [x_task(x2_ref, k) for k in range(NI)] +
             [row_task(adj1_ref, k) for k in range(NI)] +
             [row_task(adj2_ref, k) for k in range(NI)] +
             [row_task(clm_ref, k) for k in range(NI)])
    T = len(tasks)

    def start_task(t):
        if t < T:
            tasks[t][0](bufs[t % NB], sems.at[t % NB])

    def wait_task(t):
        tasks[t][1](bufs[t % NB], sems.at[t % NB])

    for t in range(NB):
        start_task(t)

    t = 0
    for tower in range(2):
        sup_scr = sup1_scr if tower == 0 else sup2_scr
        w_ref = w11_ref if tower == 0 else w21_ref
        for k in range(NI):
            wait_task(t)
            xb = bufs[t % NB][:, :F].astype(jnp.bfloat16)
            sup = jnp.dot(xb, w_ref[...], preferred_element_type=jnp.float32)
            sup_scr[pl.ds(k * BM, BM), :] = sup.astype(jnp.bfloat16)
            start_task(t + NB)
            t += 1

    for k in range(NI):
        wait_task(t)
        ab = bufs[t % NB][...].astype(jnp.bfloat16)
        adj_scr[pl.ds(k * BM, BM), :] = ab
        acc = jnp.dot(ab, sup1_scr[...], preferred_element_type=jnp.float32)
        h = jnp.maximum(acc + b11_ref[...], 0.0)
        s2 = jnp.dot(h.astype(jnp.bfloat16), w12_ref[...],
                     preferred_element_type=jnp.float32)
        s2a_scr[pl.ds(k * BM, BM), :] = s2.astype(jnp.bfloat16)
        start_task(t + NB)
        t += 1

    for k in range(NI):
        a1 = adj_scr[pl.ds(k * BM, BM), :]
        z1 = jnp.dot(a1, s2a_scr[...],
                     preferred_element_type=jnp.float32) + b12_ref[...]
        z1_ref[pl.ds(k * BM, BM), :] = z1
        r1 = jax.lax.rsqrt(jnp.sum(z1 * z1, axis=1, keepdims=True))
        sup1_scr[pl.ds(k * BM, BM), :H] = (z1 * r1 * (LOG2E / TAU)
                                           ).astype(jnp.bfloat16)
        wait_task(t)
        ab = bufs[t % NB][...].astype(jnp.bfloat16)
        adj_scr[pl.ds(k * BM, BM), :] = ab
        acc = jnp.dot(ab, sup2_scr[...], preferred_element_type=jnp.float32)
        h = jnp.maximum(acc + b21_ref[...], 0.0)
        s2 = jnp.dot(h.astype(jnp.bfloat16), w22_ref[...],
                     preferred_element_type=jnp.float32)
        s2b_scr[pl.ds(k * BM, BM), :] = s2.astype(jnp.bfloat16)
        start_task(t + NB)
        t += 1

    for k in range(NI):
        a2 = adj_scr[pl.ds(k * BM, BM), :]
        z2 = jnp.dot(a2, s2b_scr[...],
                     preferred_element_type=jnp.float32) + b22_ref[...]
        z2_ref[pl.ds(k * BM, BM), :] = z2
        r2 = jax.lax.rsqrt(jnp.sum(z2 * z2, axis=1, keepdims=True))
        sup2_scr[pl.ds(k * BM, BM), :H] = (z2 * r2).astype(jnp.bfloat16)

    for k in range(NI):
        wait_task(t)
        clm = bufs[t % NB]
        z1s = sup1_scr[pl.ds(k * BM, BM), :H]
        rs = jnp.zeros((BM, 1), dtype=jnp.float32)
        ws = jnp.zeros((BM, 1), dtype=jnp.float32)
        for half in range(2):
            z2s = sup2_scr[pl.ds(half * HC, HC), :H]
            s = jax.lax.dot_general(z1s, z2s, (((1,), (1,)), ((), ())),
                                    preferred_element_type=jnp.float32)
            pexp = jnp.exp2(s)
            rs = rs + jnp.sum(pexp, axis=1, keepdims=True)
            ws = ws + jnp.sum(pexp * clm[:, half * HC:(half + 1) * HC],
                              axis=1, keepdims=True)
        part = jnp.sum(jnp.log(rs + 1e-8) - jnp.log(ws))
        if k == 0:
            acc_ref[0] = part
        else:
            acc_ref[0] += part
        start_task(t + NB)
        t += 1

    loss_ref[...] = jnp.full((1, 1), acc_ref[0] * (1.0 / N),
                             dtype=jnp.float32)


def _cl_gcn(x1, adj1, x2, adj2, clm,
            W11, b11, W12, b12, W21, b21, W22, b22):
    z1, z2, loss = pl.pallas_call(
        _cl_gcn_kernel,
        in_specs=[
            pl.BlockSpec(memory_space=pl.ANY),
            pl.BlockSpec(memory_space=pl.ANY),
            pl.BlockSpec(memory_space=pl.ANY),
            pl.BlockSpec(memory_space=pl.ANY),
            pl.BlockSpec(memory_space=pl.ANY),
            pl.BlockSpec((F, F), lambda: (0, 0)),
            pl.BlockSpec((1, F), lambda: (0, 0)),
            pl.BlockSpec((F, H), lambda: (0, 0)),
            pl.BlockSpec((1, H), lambda: (0, 0)),
            pl.BlockSpec((F, F), lambda: (0, 0)),
            pl.BlockSpec((1, F), lambda: (0, 0)),
            pl.BlockSpec((F, H), lambda: (0, 0)),
            pl.BlockSpec((1, H), lambda: (0, 0)),
        ],
        out_specs=[
            pl.BlockSpec((N, H), lambda: (0, 0)),
            pl.BlockSpec((N, H), lambda: (0, 0)),
            pl.BlockSpec((1, 1), lambda: (0, 0)),
        ],
        out_shape=[
            jax.ShapeDtypeStruct((N, H), jnp.float32),
            jax.ShapeDtypeStruct((N, H), jnp.float32),
            jax.ShapeDtypeStruct((1, 1), jnp.float32),
        ],
        scratch_shapes=[
            pltpu.VMEM((N, N), jnp.bfloat16),
            pltpu.VMEM((N, F), jnp.bfloat16),
            pltpu.VMEM((N, F), jnp.bfloat16),
            pltpu.VMEM((N, H), jnp.bfloat16),
            pltpu.VMEM((N, H), jnp.bfloat16),
            pltpu.VMEM((BM, N), jnp.float32),
            pltpu.VMEM((BM, N), jnp.float32),
            pltpu.VMEM((BM, N), jnp.float32),
            pltpu.VMEM((BM, N), jnp.float32),
            pltpu.SemaphoreType.DMA((NB,)),
            pltpu.SMEM((1,), jnp.float32),
        ],
        compiler_params=pltpu.CompilerParams(
            vmem_limit_bytes=63 * 1024 * 1024,
        ),
    )(x1, x2, adj1, adj2, clm,
      W11.astype(jnp.bfloat16), b11.reshape(1, F),
      W12.astype(jnp.bfloat16), b12.reshape(1, H),
      W21.astype(jnp.bfloat16), b21.reshape(1, F),
      W22.astype(jnp.bfloat16), b22.reshape(1, H))
    return z1, z2, loss.reshape(())


def kernel(x1, adj1, x2, adj2, clm, W11, b11, W12, b12, W21, b21, W22, b22):
    z1, z2, loss = _cl_gcn(x1, adj1, x2, adj2, clm,
                           W11, b11, W12, b12, W21, b21, W22, b22)
    return (z1, z2, loss)
